# baseline (speedup 1.0000x reference)
import jax
import jax.numpy as jnp
from jax.experimental import pallas as pl
from jax.experimental.pallas import tpu as pltpu

N = 2048
D = 256
H = 4
DH = D // H
BI = 1024
G = N // BI
BI1 = 1024
G1 = N // BI1
CLAMP = 55.0


def _dt(x, w):
    return jax.lax.dot_general(x, w, (((1,), (1,)), ((), ())),
                               preferred_element_type=jnp.float32)


def _proj_block(st, msgw_ref, msgb_ref, wanb_ref, wacur_ref, attb_ref,
                msg_ref, b_ref, at_ref):
    lane = jax.lax.broadcasted_iota(jnp.int32, (st.shape[0], DH), 1)
    onecol = jnp.where(lane == 0, 1.0, 0.0).astype(jnp.bfloat16)
    for h in range(H):
        mh = _dt(st, msgw_ref[h]) + msgb_ref[h]
        msg_ref[h] = jnp.concatenate(
            [mh.astype(jnp.bfloat16), onecol], axis=-1)
    b_ref[...] = jnp.clip(_dt(st, wanb_ref[...]),
                          -CLAMP, CLAMP).astype(jnp.bfloat16)
    at_ref[...] = jnp.clip(jax.lax.dot_general(
        wacur_ref[...], st, (((1,), (1,)), ((), ())),
        preferred_element_type=jnp.float32) + attb_ref[...],
        -CLAMP, CLAMP).astype(jnp.bfloat16)


def _prep0_kernel(x_ref, inw_ref, inb_ref,
                  msgw_ref, msgb_ref, wanb_ref, wacur_ref, attb_ref,
                  st_ref, msg_ref, b_ref, at_ref):
    st = _dt(x_ref[...], inw_ref[...]) + inb_ref[...]
    st_ref[...] = st
    _proj_block(st, msgw_ref, msgb_ref, wanb_ref, wacur_ref, attb_ref,
                msg_ref, b_ref, at_ref)


def _attn_gru_ln(wm, msg_ref, b_ref, at_ref, st_ref,
                 wih_ref, bih_ref, whh_ref, bhh_ref, g_ref, beta_ref):
    bf = jnp.bfloat16
    wt = jnp.maximum(wm, jnp.asarray(0.0, bf))
    mbias = jnp.where(wm < jnp.asarray(0.0, bf),
                      jnp.asarray(-jnp.inf, bf), jnp.asarray(0.0, bf))
    aggs = []
    for h in range(H):
        s = b_ref[:, h:h + 1] + at_ref[h:h + 1, :]
        t = s * wt
        u = jnp.maximum(t, jnp.asarray(0.2, bf) * t)
        e = jnp.exp2(u + mbias)
        aug = jax.lax.dot_general(
            e, msg_ref[h], (((0,), (0,)), ((), ())),
            preferred_element_type=jnp.float32)
        d = jnp.maximum(aug[:, DH:DH + 1], 1e-30)
        aggs.append(aug[:, :DH] * (1.0 / d))
    agg = jnp.concatenate(aggs, axis=-1)
    st = st_ref[...]
    gi = _dt(agg, wih_ref[...]) + bih_ref[...]
    gh = _dt(st, whh_ref[...]) + bhh_ref[...]
    r = jax.nn.sigmoid(gi[:, :D] + gh[:, :D])
    z = jax.nn.sigmoid(gi[:, D:2 * D] + gh[:, D:2 * D])
    n = jnp.tanh(gi[:, 2 * D:] + r * gh[:, 2 * D:])
    new = (1.0 - z) * n + z * st
    mu = jnp.mean(new, axis=-1, keepdims=True)
    ctr = new - mu
    var = jnp.mean(ctr * ctr, axis=-1, keepdims=True)
    return ctr * jax.lax.rsqrt(var + 1e-5) * g_ref[...] + beta_ref[...]


def _layer_proj_kernel(adj_ref, w_ref, msg_ref, b_ref, at_ref, st_ref,
                       wih_ref, bih_ref, whh_ref, bhh_ref, g_ref, beta_ref,
                       msgw_ref, msgb_ref, wanb_ref, wacur_ref, attb_ref,
                       out_ref, wm_o, msg_o, b_o, at_o):
    wm = jnp.where(adj_ref[...] > 0, w_ref[...], -1.0).astype(jnp.bfloat16)
    wm_o[...] = wm
    new = _attn_gru_ln(wm, msg_ref, b_ref, at_ref, st_ref,
                       wih_ref, bih_ref, whh_ref, bhh_ref, g_ref, beta_ref)
    out_ref[...] = new
    _proj_block(new, msgw_ref, msgb_ref, wanb_ref, wacur_ref, attb_ref,
                msg_o, b_o, at_o)


def _layer_head_kernel(wm_ref, msg_ref, b_ref, at_ref, st_ref,
                       wih_ref, bih_ref, whh_ref, bhh_ref, g_ref, beta_ref,
                       w1_ref, b1_ref, w2_ref,
                       out_ref, act_ref):
    new = _attn_gru_ln(wm_ref[...], msg_ref, b_ref, at_ref, st_ref,
                       wih_ref, bih_ref, whh_ref, bhh_ref, g_ref, beta_ref)
    out_ref[...] = new
    hid = jnp.maximum(_dt(new, w1_ref[...]) + b1_ref[...], 0.0)
    act_ref[...] = jnp.sum(hid * w2_ref[...], axis=-1, keepdims=True)


def _full(shape):
    nd = len(shape)
    return pl.BlockSpec(shape, lambda i: (0,) * nd)


def _row_blk(bi=BI):
    return pl.BlockSpec((bi, D), lambda i: (i, 0))


def _proj_in_specs():
    return [_full((H, DH, D)),
            _full((H, 1, DH)),
            _full((H, D)),
            _full((8, D)),
            _full((8, 1))]


def _proj_out_specs():
    return ([pl.BlockSpec((H, BI, 2 * DH), lambda i: (0, i, 0)),
             pl.BlockSpec((BI, H), lambda i: (i, 0)),
             pl.BlockSpec((8, BI), lambda i: (0, i))],
            [jax.ShapeDtypeStruct((H, N, 2 * DH), jnp.bfloat16),
             jax.ShapeDtypeStruct((N, H), jnp.bfloat16),
             jax.ShapeDtypeStruct((8, N), jnp.bfloat16)])


def _proj_args(p):
    log2e = 1.4426950408889634
    f32 = jnp.float32
    msgw = p['msg_W'].reshape(H, DH, D)
    msgb = p['msg_b'].reshape(H, 1, DH)
    wanb = p['att_W'][:, D:] * log2e
    wacur = jnp.zeros((8, D), f32).at[:H].set(p['att_W'][:, :D] * log2e)
    attb = jnp.zeros((8, 1), f32).at[:H, 0].set(p['att_b'] * log2e)
    return msgw, msgb, wanb, wacur, attb


def _col_blk(bi=BI):
    return pl.BlockSpec((N, bi), lambda i: (0, i))


def _layer_common_specs(bi=BI):
    return [_full((H, N, 2 * DH)),
            _full((N, H)),
            pl.BlockSpec((8, bi), lambda i: (0, i)),
            _row_blk(bi),
            _full((3 * D, D)), _full((1, 3 * D)),
            _full((3 * D, D)), _full((1, 3 * D)),
            _full((1, D)), _full((1, D))]


def _layer_args(msg, bv, at, states, p):
    return (msg, bv, at, states,
            p['Wih'], p['bih'].reshape(1, 3 * D),
            p['Whh'], p['bhh'].reshape(1, 3 * D),
            p['ln_g'].reshape(1, D), p['ln_b'].reshape(1, D))


def kernel(axiom_states, adj_implies, w_implies, params):
    f32 = jnp.float32
    p0, p1 = params['layers']

    proj_outs, proj_shapes = _proj_out_specs()
    states0, msg0, bv0, at0 = pl.pallas_call(
        _prep0_kernel,
        grid=(G,),
        in_specs=[_row_blk(), _full((D, D)), _full((1, D))] + _proj_in_specs(),
        out_specs=[_row_blk()] + proj_outs,
        out_shape=[jax.ShapeDtypeStruct((N, D), f32)] + proj_shapes,
    )(axiom_states, params['in_W'], params['in_b'].reshape(1, D),
      *_proj_args(p0))

    proj_outs, proj_shapes = _proj_out_specs()
    states1, wm, msg1, bv1, at1 = pl.pallas_call(
        _layer_proj_kernel,
        grid=(G,),
        compiler_params=pltpu.CompilerParams(
            vmem_limit_bytes=100 * 1024 * 1024),
        in_specs=[_col_blk(), _col_blk()] + _layer_common_specs()
                 + _proj_in_specs(),
        out_specs=[_row_blk(), _col_blk()] + proj_outs,
        out_shape=[jax.ShapeDtypeStruct((N, D), f32),
                   jax.ShapeDtypeStruct((N, N), jnp.bfloat16)] + proj_shapes,
    )(adj_implies, w_implies,
      *_layer_args(msg0, bv0, at0, states0, p0), *_proj_args(p1))

    states2, act = pl.pallas_call(
        _layer_head_kernel,
        grid=(G1,),
        in_specs=[_col_blk(BI1)] + _layer_common_specs(BI1)
                 + [_full((D, D)), _full((1, D)), _full((1, D))],
        out_specs=[_row_blk(BI1), pl.BlockSpec((BI1, 1), lambda i: (i, 0))],
        out_shape=[jax.ShapeDtypeStruct((N, D), f32),
                   jax.ShapeDtypeStruct((N, 1), f32)],
    )(wm, *_layer_args(msg1, bv1, at1, states1, p1),
      params['out_W1'], params['out_b1'].reshape(1, D), params['out_W2'])

    return states2, act.reshape(N) + params['out_b2'][0]


# speedup vs baseline: 1.1071x; 1.1071x over previous
import jax
import jax.numpy as jnp
from jax.experimental import pallas as pl
from jax.experimental.pallas import tpu as pltpu

N = 2048
D = 256
H = 4
DH = D // H
BI = 512
G = N // BI
BI1 = 2048
G1 = N // BI1
CLAMP = 55.0


def _dt(x, w):
    return jax.lax.dot_general(x, w, (((1,), (1,)), ((), ())),
                               preferred_element_type=jnp.float32)


def _proj_block(st, msgw_ref, msgb_ref, wanb_ref, wacur_ref, attb_ref,
                msg_ref, b_ref, at_ref):
    lane = jax.lax.broadcasted_iota(jnp.int32, (st.shape[0], DH), 1)
    onecol = jnp.where(lane == 0, 1.0, 0.0).astype(jnp.bfloat16)
    for h in range(H):
        mh = _dt(st, msgw_ref[h]) + msgb_ref[h]
        msg_ref[h] = jnp.concatenate(
            [mh.astype(jnp.bfloat16), onecol], axis=-1)
    b_ref[...] = jnp.clip(_dt(st, wanb_ref[...]),
                          -CLAMP, CLAMP).astype(jnp.bfloat16)
    at_ref[...] = jnp.clip(jax.lax.dot_general(
        wacur_ref[...], st, (((1,), (1,)), ((), ())),
        preferred_element_type=jnp.float32) + attb_ref[...],
        -CLAMP, CLAMP).astype(jnp.bfloat16)


def _prep0_kernel(x_ref, inw_ref, inb_ref,
                  msgw_ref, msgb_ref, wanb_ref, wacur_ref, attb_ref,
                  st_ref, msg_ref, b_ref, at_ref):
    st = _dt(x_ref[...], inw_ref[...]) + inb_ref[...]
    st_ref[...] = st
    _proj_block(st, msgw_ref, msgb_ref, wanb_ref, wacur_ref, attb_ref,
                msg_ref, b_ref, at_ref)


def _attn_gru_ln(wm, msg_ref, b_ref, at_ref, st_ref,
                 wih_ref, bih_ref, whh_ref, bhh_ref, g_ref, beta_ref):
    bf = jnp.bfloat16
    wt = jnp.maximum(wm, jnp.asarray(0.0, bf))
    mbias = jnp.where(wm < jnp.asarray(0.0, bf),
                      jnp.asarray(-jnp.inf, bf), jnp.asarray(0.0, bf))
    aggs = []
    for h in range(H):
        s = b_ref[:, h:h + 1] + at_ref[h:h + 1, :]
        t = s * wt
        u = jnp.maximum(t, jnp.asarray(0.2, bf) * t)
        e = jnp.exp2(u + mbias)
        aug = jax.lax.dot_general(
            e, msg_ref[h], (((0,), (0,)), ((), ())),
            preferred_element_type=jnp.float32)
        d = jnp.maximum(aug[:, DH:DH + 1], 1e-30)
        aggs.append(aug[:, :DH] * (1.0 / d))
    agg = jnp.concatenate(aggs, axis=-1)
    st = st_ref[...]
    gi = _dt(agg, wih_ref[...]) + bih_ref[...]
    gh = _dt(st, whh_ref[...]) + bhh_ref[...]
    r = jax.nn.sigmoid(gi[:, :D] + gh[:, :D])
    z = jax.nn.sigmoid(gi[:, D:2 * D] + gh[:, D:2 * D])
    n = jnp.tanh(gi[:, 2 * D:] + r * gh[:, 2 * D:])
    new = (1.0 - z) * n + z * st
    mu = jnp.mean(new, axis=-1, keepdims=True)
    ctr = new - mu
    var = jnp.mean(ctr * ctr, axis=-1, keepdims=True)
    return ctr * jax.lax.rsqrt(var + 1e-5) * g_ref[...] + beta_ref[...]


def _layer_proj_kernel(adj_ref, w_ref, msg_ref, b_ref, at_ref, st_ref,
                       wih_ref, bih_ref, whh_ref, bhh_ref, g_ref, beta_ref,
                       msgw_ref, msgb_ref, wanb_ref, wacur_ref, attb_ref,
                       out_ref, wm_o, msg_o, b_o, at_o):
    wm = jnp.where(adj_ref[...] > 0, w_ref[...], -1.0).astype(jnp.bfloat16)
    wm_o[...] = wm
    new = _attn_gru_ln(wm, msg_ref, b_ref, at_ref, st_ref,
                       wih_ref, bih_ref, whh_ref, bhh_ref, g_ref, beta_ref)
    out_ref[...] = new
    _proj_block(new, msgw_ref, msgb_ref, wanb_ref, wacur_ref, attb_ref,
                msg_o, b_o, at_o)


def _layer_head_kernel(wm_ref, msg_ref, b_ref, at_ref, st_ref,
                       wih_ref, bih_ref, whh_ref, bhh_ref, g_ref, beta_ref,
                       w1_ref, b1_ref, w2_ref,
                       out_ref, act_ref):
    new = _attn_gru_ln(wm_ref[...], msg_ref, b_ref, at_ref, st_ref,
                       wih_ref, bih_ref, whh_ref, bhh_ref, g_ref, beta_ref)
    out_ref[...] = new
    hid = jnp.maximum(_dt(new, w1_ref[...]) + b1_ref[...], 0.0)
    act_ref[...] = jnp.sum(hid * w2_ref[...], axis=-1, keepdims=True)


def _full(shape):
    nd = len(shape)
    return pl.BlockSpec(shape, lambda i: (0,) * nd)


def _row_blk(bi=BI):
    return pl.BlockSpec((bi, D), lambda i: (i, 0))


def _proj_in_specs():
    return [_full((H, DH, D)),
            _full((H, 1, DH)),
            _full((H, D)),
            _full((8, D)),
            _full((8, 1))]


def _proj_out_specs():
    return ([pl.BlockSpec((H, BI, 2 * DH), lambda i: (0, i, 0)),
             pl.BlockSpec((BI, H), lambda i: (i, 0)),
             pl.BlockSpec((8, BI), lambda i: (0, i))],
            [jax.ShapeDtypeStruct((H, N, 2 * DH), jnp.bfloat16),
             jax.ShapeDtypeStruct((N, H), jnp.bfloat16),
             jax.ShapeDtypeStruct((8, N), jnp.bfloat16)])


def _proj_args(p):
    log2e = 1.4426950408889634
    f32 = jnp.float32
    msgw = p['msg_W'].reshape(H, DH, D)
    msgb = p['msg_b'].reshape(H, 1, DH)
    wanb = p['att_W'][:, D:] * log2e
    wacur = jnp.zeros((8, D), f32).at[:H].set(p['att_W'][:, :D] * log2e)
    attb = jnp.zeros((8, 1), f32).at[:H, 0].set(p['att_b'] * log2e)
    return msgw, msgb, wanb, wacur, attb


def _col_blk(bi=BI):
    return pl.BlockSpec((N, bi), lambda i: (0, i))


def _layer_common_specs(bi=BI):
    return [_full((H, N, 2 * DH)),
            _full((N, H)),
            pl.BlockSpec((8, bi), lambda i: (0, i)),
            _row_blk(bi),
            _full((3 * D, D)), _full((1, 3 * D)),
            _full((3 * D, D)), _full((1, 3 * D)),
            _full((1, D)), _full((1, D))]


def _layer_args(msg, bv, at, states, p):
    return (msg, bv, at, states,
            p['Wih'], p['bih'].reshape(1, 3 * D),
            p['Whh'], p['bhh'].reshape(1, 3 * D),
            p['ln_g'].reshape(1, D), p['ln_b'].reshape(1, D))


def kernel(axiom_states, adj_implies, w_implies, params):
    f32 = jnp.float32
    p0, p1 = params['layers']

    proj_outs, proj_shapes = _proj_out_specs()
    states0, msg0, bv0, at0 = pl.pallas_call(
        _prep0_kernel,
        grid=(G,),
        in_specs=[_row_blk(), _full((D, D)), _full((1, D))] + _proj_in_specs(),
        out_specs=[_row_blk()] + proj_outs,
        out_shape=[jax.ShapeDtypeStruct((N, D), f32)] + proj_shapes,
    )(axiom_states, params['in_W'], params['in_b'].reshape(1, D),
      *_proj_args(p0))

    proj_outs, proj_shapes = _proj_out_specs()
    states1, wm, msg1, bv1, at1 = pl.pallas_call(
        _layer_proj_kernel,
        grid=(G,),
        in_specs=[_col_blk(), _col_blk()] + _layer_common_specs()
                 + _proj_in_specs(),
        out_specs=[_row_blk(), _col_blk()] + proj_outs,
        out_shape=[jax.ShapeDtypeStruct((N, D), f32),
                   jax.ShapeDtypeStruct((N, N), jnp.bfloat16)] + proj_shapes,
    )(adj_implies, w_implies,
      *_layer_args(msg0, bv0, at0, states0, p0), *_proj_args(p1))

    states2, act = pl.pallas_call(
        _layer_head_kernel,
        grid=(G1,),
        in_specs=[_col_blk(BI1)] + _layer_common_specs(BI1)
                 + [_full((D, D)), _full((1, D)), _full((1, D))],
        out_specs=[_row_blk(BI1), pl.BlockSpec((BI1, 1), lambda i: (i, 0))],
        out_shape=[jax.ShapeDtypeStruct((N, D), f32),
                   jax.ShapeDtypeStruct((N, 1), f32)],
    )(wm, *_layer_args(msg1, bv1, at1, states1, p1),
      params['out_W1'], params['out_b1'].reshape(1, D), params['out_W2'])

    return states2, act.reshape(N) + params['out_b2'][0]


# speedup vs baseline: 1.1189x; 1.0106x over previous
import jax
import jax.numpy as jnp
from jax.experimental import pallas as pl
from jax.experimental.pallas import tpu as pltpu

N = 2048
D = 256
H = 4
DH = D // H
BI = 512
G = N // BI
BI1 = 2048
G1 = N // BI1
CLAMP = 55.0


def _dt(x, w):
    return jax.lax.dot_general(x, w, (((1,), (1,)), ((), ())),
                               preferred_element_type=jnp.float32)


def _proj_block(st, msgw_ref, msgb_ref, wanb_ref, wacur_ref, attb_ref,
                msg_ref, b_ref, at_ref):
    lane = jax.lax.broadcasted_iota(jnp.int32, (st.shape[0], DH), 1)
    onecol = jnp.where(lane == 0, 1.0, 0.0).astype(jnp.bfloat16)
    for h in range(H):
        mh = _dt(st, msgw_ref[h]) + msgb_ref[h]
        msg_ref[h] = jnp.concatenate(
            [mh.astype(jnp.bfloat16), onecol], axis=-1)
    b_ref[...] = jnp.clip(_dt(st, wanb_ref[...]),
                          -CLAMP, CLAMP).astype(jnp.bfloat16)
    at_ref[...] = jnp.clip(jax.lax.dot_general(
        wacur_ref[...], st, (((1,), (1,)), ((), ())),
        preferred_element_type=jnp.float32) + attb_ref[...],
        -CLAMP, CLAMP).astype(jnp.bfloat16)


def _prep0_kernel(x_ref, inw_ref, inb_ref,
                  msgw_ref, msgb_ref, wanb_ref, wacur_ref, attb_ref,
                  st_ref, msg_ref, b_ref, at_ref):
    st = _dt(x_ref[...], inw_ref[...]) + inb_ref[...]
    st_ref[...] = st
    _proj_block(st, msgw_ref, msgb_ref, wanb_ref, wacur_ref, attb_ref,
                msg_ref, b_ref, at_ref)


def _attn_gru_ln(wm, msg_ref, b_ref, at_ref, st_ref,
                 wih_ref, bih_ref, whh_ref, bhh_ref, g_ref, beta_ref):
    bf = jnp.bfloat16
    wt = jnp.maximum(wm, jnp.asarray(0.0, bf))
    mbias = jnp.where(wm < jnp.asarray(0.0, bf),
                      jnp.asarray(-jnp.inf, bf), jnp.asarray(0.0, bf))
    aggs = []
    for h in range(H):
        s = b_ref[:, h:h + 1] + at_ref[h:h + 1, :]
        t = s * wt
        u = jnp.maximum(t, jnp.asarray(0.2, bf) * t)
        e = jnp.exp2(u + mbias)
        aug = jax.lax.dot_general(
            e, msg_ref[h], (((0,), (0,)), ((), ())),
            preferred_element_type=jnp.float32)
        d = jnp.maximum(aug[:, DH:DH + 1], 1e-30)
        aggs.append(aug[:, :DH] * (1.0 / d))
    agg = jnp.concatenate(aggs, axis=-1)
    st = st_ref[...]
    gi = _dt(agg, wih_ref[...]) + bih_ref[...]
    gh = _dt(st, whh_ref[...]) + bhh_ref[...]
    r = jax.nn.sigmoid(gi[:, :D] + gh[:, :D])
    z = jax.nn.sigmoid(gi[:, D:2 * D] + gh[:, D:2 * D])
    n = jnp.tanh(gi[:, 2 * D:] + r * gh[:, 2 * D:])
    new = (1.0 - z) * n + z * st
    mu = jnp.mean(new, axis=-1, keepdims=True)
    ctr = new - mu
    var = jnp.mean(ctr * ctr, axis=-1, keepdims=True)
    return ctr * jax.lax.rsqrt(var + 1e-5) * g_ref[...] + beta_ref[...]


def _layer_proj_kernel(adj_ref, w_ref, msg_ref, b_ref, at_ref, st_ref,
                       wih_ref, bih_ref, whh_ref, bhh_ref, g_ref, beta_ref,
                       msgw_ref, msgb_ref, wanb_ref, wacur_ref, attb_ref,
                       out_ref, wm_o, msg_o, b_o, at_o):
    wm = jnp.where(adj_ref[...] > 0, w_ref[...], -1.0).astype(jnp.bfloat16)
    wm_o[...] = wm
    new = _attn_gru_ln(wm, msg_ref, b_ref, at_ref, st_ref,
                       wih_ref, bih_ref, whh_ref, bhh_ref, g_ref, beta_ref)
    out_ref[...] = new
    _proj_block(new, msgw_ref, msgb_ref, wanb_ref, wacur_ref, attb_ref,
                msg_o, b_o, at_o)


def _layer_head_kernel(wm_ref, msg_ref, b_ref, at_ref, st_ref,
                       wih_ref, bih_ref, whh_ref, bhh_ref, g_ref, beta_ref,
                       w1_ref, b1_ref, w2_ref,
                       out_ref, act_ref):
    new = _attn_gru_ln(wm_ref[...], msg_ref, b_ref, at_ref, st_ref,
                       wih_ref, bih_ref, whh_ref, bhh_ref, g_ref, beta_ref)
    out_ref[...] = new
    hid = jnp.maximum(_dt(new, w1_ref[...]) + b1_ref[...], 0.0)
    act_ref[...] = jnp.sum(hid * w2_ref[...], axis=-1, keepdims=True)


def _full(shape):
    nd = len(shape)
    return pl.BlockSpec(shape, lambda i: (0,) * nd)


def _row_blk(bi=BI):
    return pl.BlockSpec((bi, D), lambda i: (i, 0))


def _proj_in_specs():
    return [_full((H, DH, D)),
            _full((H, 1, DH)),
            _full((H, D)),
            _full((8, D)),
            _full((8, 1))]


def _proj_out_specs(bi=BI):
    return ([pl.BlockSpec((H, bi, 2 * DH), lambda i: (0, i, 0)),
             pl.BlockSpec((bi, H), lambda i: (i, 0)),
             pl.BlockSpec((8, bi), lambda i: (0, i))],
            [jax.ShapeDtypeStruct((H, N, 2 * DH), jnp.bfloat16),
             jax.ShapeDtypeStruct((N, H), jnp.bfloat16),
             jax.ShapeDtypeStruct((8, N), jnp.bfloat16)])


def _proj_args(p):
    log2e = 1.4426950408889634
    f32 = jnp.float32
    msgw = p['msg_W'].reshape(H, DH, D)
    msgb = p['msg_b'].reshape(H, 1, DH)
    wanb = p['att_W'][:, D:] * log2e
    wacur = jnp.zeros((8, D), f32).at[:H].set(p['att_W'][:, :D] * log2e)
    attb = jnp.zeros((8, 1), f32).at[:H, 0].set(p['att_b'] * log2e)
    return msgw, msgb, wanb, wacur, attb


def _col_blk(bi=BI):
    return pl.BlockSpec((N, bi), lambda i: (0, i))


def _layer_common_specs(bi=BI):
    return [_full((H, N, 2 * DH)),
            _full((N, H)),
            pl.BlockSpec((8, bi), lambda i: (0, i)),
            _row_blk(bi),
            _full((3 * D, D)), _full((1, 3 * D)),
            _full((3 * D, D)), _full((1, 3 * D)),
            _full((1, D)), _full((1, D))]


def _layer_args(msg, bv, at, states, p):
    return (msg, bv, at, states,
            p['Wih'], p['bih'].reshape(1, 3 * D),
            p['Whh'], p['bhh'].reshape(1, 3 * D),
            p['ln_g'].reshape(1, D), p['ln_b'].reshape(1, D))


def kernel(axiom_states, adj_implies, w_implies, params):
    f32 = jnp.float32
    p0, p1 = params['layers']

    proj_outs, proj_shapes = _proj_out_specs(N)
    states0, msg0, bv0, at0 = pl.pallas_call(
        _prep0_kernel,
        grid=(1,),
        in_specs=[_row_blk(N), _full((D, D)), _full((1, D))]
                 + _proj_in_specs(),
        out_specs=[_row_blk(N)] + proj_outs,
        out_shape=[jax.ShapeDtypeStruct((N, D), f32)] + proj_shapes,
    )(axiom_states, params['in_W'], params['in_b'].reshape(1, D),
      *_proj_args(p0))

    proj_outs, proj_shapes = _proj_out_specs()
    states1, wm, msg1, bv1, at1 = pl.pallas_call(
        _layer_proj_kernel,
        grid=(G,),
        in_specs=[_col_blk(), _col_blk()] + _layer_common_specs()
                 + _proj_in_specs(),
        out_specs=[_row_blk(), _col_blk()] + proj_outs,
        out_shape=[jax.ShapeDtypeStruct((N, D), f32),
                   jax.ShapeDtypeStruct((N, N), jnp.bfloat16)] + proj_shapes,
    )(adj_implies, w_implies,
      *_layer_args(msg0, bv0, at0, states0, p0), *_proj_args(p1))

    states2, act = pl.pallas_call(
        _layer_head_kernel,
        grid=(G1,),
        in_specs=[_col_blk(BI1)] + _layer_common_specs(BI1)
                 + [_full((D, D)), _full((1, D)), _full((1, D))],
        out_specs=[_row_blk(BI1), pl.BlockSpec((BI1, 1), lambda i: (i, 0))],
        out_shape=[jax.ShapeDtypeStruct((N, D), f32),
                   jax.ShapeDtypeStruct((N, 1), f32)],
    )(wm, *_layer_args(msg1, bv1, at1, states1, p1),
      params['out_W1'], params['out_b1'].reshape(1, D), params['out_W2'])

    return states2, act.reshape(N) + params['out_b2'][0]


# speedup vs baseline: 1.1196x; 1.0006x over previous
import jax
import jax.numpy as jnp
from jax.experimental import pallas as pl

N = 2048
D = 256
H = 4
DH = D // H
BI = 512
G = N // BI
BI1 = 2048
G1 = N // BI1
CLAMP = 55.0


def _dt(x, w):
    return jax.lax.dot_general(x, w, (((1,), (1,)), ((), ())),
                               preferred_element_type=jnp.float32)


def _proj_block(st, msgw_ref, msgb_ref, wanb_ref, wacur_ref, attb_ref,
                msg_ref, b_ref, at_ref):
    lane = jax.lax.broadcasted_iota(jnp.int32, (st.shape[0], DH), 1)
    onecol = jnp.where(lane == 0, 1.0, 0.0).astype(jnp.bfloat16)
    for h in range(H):
        mh = _dt(st, msgw_ref[h]) + msgb_ref[h]
        msg_ref[h] = jnp.concatenate(
            [mh.astype(jnp.bfloat16), onecol], axis=-1)
    b_ref[...] = jnp.clip(_dt(st, wanb_ref[...]),
                          -CLAMP, CLAMP).astype(jnp.bfloat16)
    at_ref[...] = jnp.clip(jax.lax.dot_general(
        wacur_ref[...], st, (((1,), (1,)), ((), ())),
        preferred_element_type=jnp.float32) + attb_ref[...],
        -CLAMP, CLAMP).astype(jnp.bfloat16)


def _prep0_kernel(x_ref, inw_ref, inb_ref,
                  msgw_ref, msgb_ref, wanb_ref, wacur_ref, attb_ref,
                  st_ref, msg_ref, b_ref, at_ref):
    st = _dt(x_ref[...], inw_ref[...]) + inb_ref[...]
    st_ref[...] = st
    _proj_block(st, msgw_ref, msgb_ref, wanb_ref, wacur_ref, attb_ref,
                msg_ref, b_ref, at_ref)


def _attn_gru_ln(wm, msg_ref, b_ref, at_ref, st_ref,
                 wih_ref, bih_ref, whh_ref, bhh_ref, g_ref, beta_ref):
    bf = jnp.bfloat16
    wt = jnp.maximum(wm, jnp.asarray(0.0, bf))
    mbias = jnp.where(wm < jnp.asarray(0.0, bf),
                      jnp.asarray(-jnp.inf, bf), jnp.asarray(0.0, bf))
    aggs = []
    for h in range(H):
        s = b_ref[:, h:h + 1] + at_ref[h:h + 1, :]
        t = s * wt
        u = jnp.maximum(t, jnp.asarray(0.2, bf) * t)
        e = jnp.exp2(u + mbias)
        aug = jax.lax.dot_general(
            e, msg_ref[h], (((0,), (0,)), ((), ())),
            preferred_element_type=jnp.float32)
        d = jnp.maximum(aug[:, DH:DH + 1], 1e-30)
        aggs.append(aug[:, :DH] * (1.0 / d))
    agg = jnp.concatenate(aggs, axis=-1)
    st = st_ref[...]
    gi = _dt(agg, wih_ref[...]) + bih_ref[...]
    gh = _dt(st, whh_ref[...]) + bhh_ref[...]
    r = jax.nn.sigmoid(gi[:, :D] + gh[:, :D])
    z = jax.nn.sigmoid(gi[:, D:2 * D] + gh[:, D:2 * D])
    n = jnp.tanh(gi[:, 2 * D:] + r * gh[:, 2 * D:])
    new = (1.0 - z) * n + z * st
    mu = jnp.mean(new, axis=-1, keepdims=True)
    ctr = new - mu
    var = jnp.mean(ctr * ctr, axis=-1, keepdims=True)
    return ctr * jax.lax.rsqrt(var + 1e-5) * g_ref[...] + beta_ref[...]


def _layer_proj_kernel(adj_ref, w_ref, msg_ref, b_ref, at_ref, st_ref,
                       wih_ref, bih_ref, whh_ref, bhh_ref, g_ref, beta_ref,
                       msgw_ref, msgb_ref, wanb_ref, wacur_ref, attb_ref,
                       out_ref, wm_o, msg_o, b_o, at_o):
    wm = jnp.where(adj_ref[...] > 0, w_ref[...], -1.0).astype(jnp.bfloat16)
    wm_o[...] = wm
    new = _attn_gru_ln(wm, msg_ref, b_ref, at_ref, st_ref,
                       wih_ref, bih_ref, whh_ref, bhh_ref, g_ref, beta_ref)
    out_ref[...] = new
    _proj_block(new, msgw_ref, msgb_ref, wanb_ref, wacur_ref, attb_ref,
                msg_o, b_o, at_o)


def _layer_head_kernel(wm_ref, msg_ref, b_ref, at_ref, st_ref,
                       wih_ref, bih_ref, whh_ref, bhh_ref, g_ref, beta_ref,
                       w1_ref, b1_ref, w2_ref,
                       out_ref, act_ref):
    new = _attn_gru_ln(wm_ref[...], msg_ref, b_ref, at_ref, st_ref,
                       wih_ref, bih_ref, whh_ref, bhh_ref, g_ref, beta_ref)
    out_ref[...] = new
    hid = jnp.maximum(_dt(new, w1_ref[...]) + b1_ref[...], 0.0)
    act_ref[...] = jnp.sum(hid * w2_ref[...], axis=-1, keepdims=True)


def _full(shape):
    nd = len(shape)
    return pl.BlockSpec(shape, lambda i: (0,) * nd)


def _row_blk(bi=BI):
    return pl.BlockSpec((bi, D), lambda i: (i, 0))


def _proj_in_specs():
    return [_full((H, DH, D)),
            _full((H, 1, DH)),
            _full((H, D)),
            _full((8, D)),
            _full((8, 1))]


def _proj_out_specs(bi=BI):
    return ([pl.BlockSpec((H, bi, 2 * DH), lambda i: (0, i, 0)),
             pl.BlockSpec((bi, H), lambda i: (i, 0)),
             pl.BlockSpec((8, bi), lambda i: (0, i))],
            [jax.ShapeDtypeStruct((H, N, 2 * DH), jnp.bfloat16),
             jax.ShapeDtypeStruct((N, H), jnp.bfloat16),
             jax.ShapeDtypeStruct((8, N), jnp.bfloat16)])


def _proj_args(p):
    log2e = 1.4426950408889634
    f32 = jnp.float32
    msgw = p['msg_W'].reshape(H, DH, D)
    msgb = p['msg_b'].reshape(H, 1, DH)
    wanb = p['att_W'][:, D:] * log2e
    wacur = jnp.zeros((8, D), f32).at[:H].set(p['att_W'][:, :D] * log2e)
    attb = jnp.zeros((8, 1), f32).at[:H, 0].set(p['att_b'] * log2e)
    return msgw, msgb, wanb, wacur, attb


def _col_blk(bi=BI):
    return pl.BlockSpec((N, bi), lambda i: (0, i))


def _layer_common_specs(bi=BI):
    return [_full((H, N, 2 * DH)),
            _full((N, H)),
            pl.BlockSpec((8, bi), lambda i: (0, i)),
            _row_blk(bi),
            _full((3 * D, D)), _full((1, 3 * D)),
            _full((3 * D, D)), _full((1, 3 * D)),
            _full((1, D)), _full((1, D))]


def _layer_args(msg, bv, at, states, p):
    return (msg, bv, at, states,
            p['Wih'], p['bih'].reshape(1, 3 * D),
            p['Whh'], p['bhh'].reshape(1, 3 * D),
            p['ln_g'].reshape(1, D), p['ln_b'].reshape(1, D))


def kernel(axiom_states, adj_implies, w_implies, params):
    f32 = jnp.float32
    p0, p1 = params['layers']

    proj_outs, proj_shapes = _proj_out_specs(N)
    states0, msg0, bv0, at0 = pl.pallas_call(
        _prep0_kernel,
        grid=(1,),
        in_specs=[_row_blk(N), _full((D, D)), _full((1, D))]
                 + _proj_in_specs(),
        out_specs=[_row_blk(N)] + proj_outs,
        out_shape=[jax.ShapeDtypeStruct((N, D), f32)] + proj_shapes,
    )(axiom_states, params['in_W'], params['in_b'].reshape(1, D),
      *_proj_args(p0))

    proj_outs, proj_shapes = _proj_out_specs()
    states1, wm, msg1, bv1, at1 = pl.pallas_call(
        _layer_proj_kernel,
        grid=(G,),
        in_specs=[_col_blk(), _col_blk()] + _layer_common_specs()
                 + _proj_in_specs(),
        out_specs=[_row_blk(), _col_blk()] + proj_outs,
        out_shape=[jax.ShapeDtypeStruct((N, D), f32),
                   jax.ShapeDtypeStruct((N, N), jnp.bfloat16)] + proj_shapes,
    )(adj_implies, w_implies,
      *_layer_args(msg0, bv0, at0, states0, p0), *_proj_args(p1))

    states2, act = pl.pallas_call(
        _layer_head_kernel,
        grid=(G1,),
        in_specs=[_col_blk(BI1)] + _layer_common_specs(BI1)
                 + [_full((D, D)), _full((1, D)), _full((1, D))],
        out_specs=[_row_blk(BI1), pl.BlockSpec((BI1, 1), lambda i: (i, 0))],
        out_shape=[jax.ShapeDtypeStruct((N, D), f32),
                   jax.ShapeDtypeStruct((N, 1), f32)],
    )(wm, *_layer_args(msg1, bv1, at1, states1, p1),
      params['out_W1'], params['out_b1'].reshape(1, D), params['out_W2'])

    return states2, act.reshape(N) + params['out_b2'][0]
